# baseline (speedup 1.0000x reference)
import jax
import jax.numpy as jnp
from jax.experimental import pallas as pl
from jax.experimental.pallas import tpu as pltpu
from functools import partial

_EPS = 1e-05


def _gn_kernel(x_ref, gid_ref, gamma_ref, beta_ref, o_ref, *, hw, G):
    x = x_ref[...]
    C = x.shape[1]
    ids = gid_ref[...]
    MT = (jax.lax.broadcasted_iota(jnp.int32, (G, C), 0) == ids).astype(x.dtype)
    s = jnp.sum(x, axis=(2, 3))
    ss = jnp.sum(x * x, axis=(2, 3))
    dn = (((1,), (1,)), ((), ()))
    gs = jax.lax.dot_general(s, MT, dn)
    gss = jax.lax.dot_general(ss, MT, dn)
    cnt = jnp.sum(MT, axis=1)[None, :]
    n = cnt * hw
    mean = gs / n
    var = (gss - n * mean * mean) / (n - 1.0)
    inv = jax.lax.rsqrt(var + _EPS)
    mean_c = jnp.dot(mean, MT)
    inv_c = jnp.dot(inv, MT)
    scale = inv_c * gamma_ref[...]
    shift = beta_ref[...] - mean_c * scale
    o_ref[...] = x * scale[:, :, None, None] + shift[:, :, None, None]


def kernel(x, gamma, beta, group_ids):
    B, C, H, W = x.shape
    G = 8
    BB = 2
    gid2 = group_ids.reshape(1, C)
    gamma2 = gamma.reshape(1, C)
    beta2 = beta.reshape(1, C)

    return pl.pallas_call(
        partial(_gn_kernel, hw=float(H * W), G=G),
        grid=(B // BB,),
        in_specs=[
            pl.BlockSpec((BB, C, H, W), lambda b: (b, 0, 0, 0)),
            pl.BlockSpec((1, C), lambda b: (0, 0)),
            pl.BlockSpec((1, C), lambda b: (0, 0)),
            pl.BlockSpec((1, C), lambda b: (0, 0)),
        ],
        out_specs=pl.BlockSpec((BB, C, H, W), lambda b: (b, 0, 0, 0)),
        out_shape=jax.ShapeDtypeStruct((B, C, H, W), x.dtype),
        compiler_params=pltpu.CompilerParams(
            dimension_semantics=("parallel",),
            vmem_limit_bytes=100 * 1024 * 1024,
        ),
    )(x, gid2, gamma2, beta2)
